# baseline (speedup 1.0000x reference)
import functools

import jax
import jax.numpy as jnp
from jax import lax
from jax.experimental import pallas as pl
from jax.experimental.pallas import tpu as pltpu
from jax.experimental.pallas import tpu_sc as plsc

B = 4096
L = 200
DIM = 64
N = B * L
ROW = 128
NROWS = N // ROW
NC = 2
NS = 16
NW = NC * NS
ROWS_PER_W = NROWS // NW
CH = 8
NCHUNK = ROWS_PER_W // CH


@functools.partial(jax.jit, static_argnames=())
def _embed(table, idx):
    mesh = plsc.VectorSubcoreMesh(core_axis_name="c", subcore_axis_name="s")

    @functools.partial(
        pl.kernel,
        mesh=mesh,
        out_type=jax.ShapeDtypeStruct((NROWS, ROW, DIM), jnp.float32),
        scratch_types=[
            pltpu.VMEM((CH, ROW), jnp.int32),
            pltpu.VMEM((CH, ROW, DIM), jnp.float32),
            pltpu.SemaphoreType.DMA,
        ],
        compiler_params=pltpu.CompilerParams(use_tc_tiling_on_sc=False),
    )
    def emb(table_hbm, idx_hbm, out_hbm, idx_v, rows_v, sem):
        wid = lax.axis_index("s") * NC + lax.axis_index("c")
        row_base = wid * ROWS_PER_W

        def body(g, carry):
            r0 = row_base + g * CH
            pltpu.sync_copy(idx_hbm.at[pl.ds(r0, CH)], idx_v)
            copies = [
                pltpu.async_copy(table_hbm.at[idx_v.at[j]], rows_v.at[j], sem)
                for j in range(CH)
            ]
            for c in copies:
                c.wait()
            pltpu.sync_copy(rows_v, out_hbm.at[pl.ds(r0, CH)])
            return carry

        lax.fori_loop(0, NCHUNK, body, 0)

    return emb(table, idx)


def kernel(x, table):
    idx = x.reshape(NROWS, ROW)
    out = _embed(table, idx)
    return out.reshape(B, L, DIM)


# speedup vs baseline: 1.0070x; 1.0070x over previous
import functools

import jax
import jax.numpy as jnp
from jax import lax
from jax.experimental import pallas as pl
from jax.experimental.pallas import tpu as pltpu
from jax.experimental.pallas import tpu_sc as plsc

B = 4096
L = 200
DIM = 64
N = B * L
ROW = 128
NROWS = N // ROW
NC = 2
NS = 16
NW = NC * NS
ROWS_PER_W = NROWS // NW
NBUF = 2
CH = 4
NCHUNK = ROWS_PER_W // CH
NITER = NCHUNK // NBUF


@jax.jit
def _embed(table, idx):
    mesh = plsc.VectorSubcoreMesh(core_axis_name="c", subcore_axis_name="s")

    @functools.partial(
        pl.kernel,
        mesh=mesh,
        out_type=jax.ShapeDtypeStruct((NROWS, ROW, DIM), jnp.float32),
        scratch_types=[
            pltpu.VMEM((NBUF, CH, ROW), jnp.int32),
            pltpu.VMEM((NBUF, CH, ROW, DIM), jnp.float32),
            pltpu.SemaphoreType.DMA,
            pltpu.SemaphoreType.DMA,
            pltpu.SemaphoreType.DMA,
            pltpu.SemaphoreType.DMA,
        ],
        compiler_params=pltpu.CompilerParams(use_tc_tiling_on_sc=False),
    )
    def emb(table_hbm, idx_hbm, out_hbm, idx_v, rows_v, g0, g1, s0, s1):
        gsem = (g0, g1)
        ssem = (s0, s1)
        wid = lax.axis_index("s") * NC + lax.axis_index("c")
        row_base = wid * ROWS_PER_W

        def fire_gathers(c, b):
            r0 = row_base + c * CH
            pltpu.sync_copy(idx_hbm.at[pl.ds(r0, CH)], idx_v.at[b])
            for j in range(CH):
                pltpu.async_copy(
                    table_hbm.at[idx_v.at[b].at[j]], rows_v.at[b].at[j], gsem[b]
                )

        def wait_gathers(b):
            for j in range(CH):
                pltpu.make_async_copy(
                    table_hbm.at[idx_v.at[b].at[j]], rows_v.at[b].at[j], gsem[b]
                ).wait()

        for b in range(NBUF):
            fire_gathers(b, b)

        def body(g, carry):
            for b in range(NBUF):
                c = g * NBUF + b
                r0 = row_base + c * CH
                wait_gathers(b)
                st = pltpu.async_copy(
                    rows_v.at[b], out_hbm.at[pl.ds(r0, CH)], ssem[b]
                )
                st.wait()
                nxt = c + NBUF

                @pl.when(nxt < NCHUNK)
                def _():
                    fire_gathers(nxt, b)

            return carry

        lax.fori_loop(0, NITER, body, 0)

    return emb(table, idx)


def kernel(x, table):
    idx = x.reshape(NROWS, ROW)
    out = _embed(table, idx)
    return out.reshape(B, L, DIM)
